# baseline (speedup 1.0000x reference)
import functools

import jax
import jax.numpy as jnp
from jax import lax
from jax.experimental import pallas as pl
from jax.experimental.pallas import tpu as pltpu
from jax.experimental.pallas import tpu_sc as plsc

VOCAB = 100000
D_MODEL = 1024
MAX_POS = 2048
BATCH = 4
SEQ = 2048
EPS = 1e-05

NC = 2
NS = 16
NW = NC * NS
G = 32
S_PER_W = SEQ // NW
NBLK = D_MODEL // 128
UNROLL = 8
ROWS = BATCH * SEQ


def _rsqrt(x):
    i = lax.bitcast_convert_type(x, jnp.int32)
    i = jnp.int32(0x5F3759DF) - lax.shift_right_arithmetic(i, 1)
    y = lax.bitcast_convert_type(i, jnp.float32)
    for _ in range(3):
        y = y * (jnp.float32(1.5) - jnp.float32(0.5) * x * y * y)
    return y


_GATHER_DNUMS = lax.GatherDimensionNumbers(
    offset_dims=(), collapsed_slice_dims=(0,), start_index_map=(0,))


def _take16(v, idx):
    return lax.gather(v, idx[:, None], _GATHER_DNUMS, (1,),
                      mode=lax.GatherScatterMode.PROMISE_IN_BOUNDS)


def _lane_sum(v):
    lanes = lax.iota(jnp.int32, 16)
    for sh in (8, 4, 2, 1):
        v = v + _take16(v, (lanes + sh) & 15)
    return v


def _sc_body(ids_hbm, wemb_hbm, pemb_hbm, gamma_hbm, beta_hbm, out_hbm,
             idx0, idx1, rows0, rows1, pos_v, gamma_v, beta_v, mscr, rscr,
             gsem0, gsem1, osem0, osem1):
    wid = lax.axis_index("s") * NC + lax.axis_index("c")
    s0 = wid * S_PER_W

    pltpu.sync_copy(gamma_hbm, gamma_v)
    pltpu.sync_copy(beta_hbm, beta_v)

    def gbase(g):
        return (g & 3) * SEQ + s0 + (g >> 2) * G

    def compute(rows_v):
        z = jnp.zeros((16,), jnp.float32)

        def row_sum(r):
            def col_sum(i, carry4):
                a, a2 = carry4
                d = i * 16
                x = rows_v[r, pl.ds(d, 16)] + pos_v[r, pl.ds(d, 16)]
                rows_v[r, pl.ds(d, 16)] = x
                return a + x, a2 + x * x

            acc, acc2 = plsc.parallel_loop(
                0, D_MODEL // 16, unroll=UNROLL, carry=(z, z))(col_sum)
            s1 = _lane_sum(acc)
            s2 = _lane_sum(acc2)
            mean = s1 * jnp.float32(1.0 / D_MODEL)
            var = s2 * jnp.float32(1.0 / D_MODEL) - mean * mean
            mscr[r, pl.ds(0, 16)] = mean
            rscr[r, pl.ds(0, 16)] = _rsqrt(var + jnp.float32(EPS))

        plsc.parallel_loop(0, G)(row_sum)

        for kb in range(NBLK):
            gv = [gamma_v[pl.ds(kb * 128 + u * 16, 16)] for u in range(8)]
            bv = [beta_v[pl.ds(kb * 128 + u * 16, 16)] for u in range(8)]

            def row_norm(r, gv=gv, bv=bv, kb=kb):
                mean_v = mscr[r, pl.ds(0, 16)]
                rstd_v = rscr[r, pl.ds(0, 16)]
                for u in range(8):
                    d = kb * 128 + u * 16
                    x = rows_v[r, pl.ds(d, 16)]
                    rows_v[r, pl.ds(d, 16)] = (
                        (x - mean_v) * rstd_v * gv[u] + bv[u])

            plsc.parallel_loop(0, G, unroll=4)(row_norm)

    pltpu.sync_copy(pemb_hbm.at[pl.ds(s0, G)], pos_v)
    pltpu.sync_copy(ids_hbm.at[pl.ds(s0, G)], idx0)
    pltpu.async_copy(wemb_hbm.at[idx0], rows0, gsem0)

    def pipe(t, carry):
        g0 = 2 * t
        g1 = 2 * t + 1
        base0 = gbase(g0)
        base1 = gbase(g1)

        @pl.when(t == 2)
        def _():
            pltpu.sync_copy(pemb_hbm.at[pl.ds(s0 + G, G)], pos_v)

        pltpu.make_async_copy(wemb_hbm.at[idx0], rows0, gsem0).wait()

        @pl.when(t > 0)
        def _():
            pltpu.make_async_copy(rows1, out_hbm.at[pl.ds(base1, G)],
                                  osem1).wait()

        pltpu.sync_copy(ids_hbm.at[pl.ds(base1, G)], idx1)
        pltpu.async_copy(wemb_hbm.at[idx1], rows1, gsem1)

        compute(rows0)
        pltpu.async_copy(rows0, out_hbm.at[pl.ds(base0, G)], osem0)

        pltpu.make_async_copy(wemb_hbm.at[idx1], rows1, gsem1).wait()

        @pl.when(t < 3)
        def _():
            base2 = gbase(g0 + 2)
            pltpu.make_async_copy(rows0, out_hbm.at[pl.ds(base2, G)],
                                  osem0).wait()
            pltpu.sync_copy(ids_hbm.at[pl.ds(base2, G)], idx0)
            pltpu.async_copy(wemb_hbm.at[idx0], rows0, gsem0)

        compute(rows1)
        pltpu.async_copy(rows1, out_hbm.at[pl.ds(base1, G)], osem1)
        return carry

    lax.fori_loop(0, BATCH, pipe, 0)

    last0 = gbase(jnp.int32(6))
    last1 = gbase(jnp.int32(7))
    pltpu.make_async_copy(rows0, out_hbm.at[pl.ds(last0, G)], osem0).wait()
    pltpu.make_async_copy(rows1, out_hbm.at[pl.ds(last1, G)], osem1).wait()


@jax.jit
def _run(ids_flat, word_emb, pos_emb, gamma, beta):
    mesh = plsc.VectorSubcoreMesh(core_axis_name="c", subcore_axis_name="s")
    k = functools.partial(
        pl.kernel,
        out_type=jax.ShapeDtypeStruct((ROWS, D_MODEL), jnp.float32),
        mesh=mesh,
        scratch_types=[
            pltpu.VMEM((G,), jnp.int32),
            pltpu.VMEM((G,), jnp.int32),
            pltpu.VMEM((G, D_MODEL), jnp.float32),
            pltpu.VMEM((G, D_MODEL), jnp.float32),
            pltpu.VMEM((G, D_MODEL), jnp.float32),
            pltpu.VMEM((D_MODEL,), jnp.float32),
            pltpu.VMEM((D_MODEL,), jnp.float32),
            pltpu.VMEM((G, 16), jnp.float32),
            pltpu.VMEM((G, 16), jnp.float32),
            pltpu.SemaphoreType.DMA,
            pltpu.SemaphoreType.DMA,
            pltpu.SemaphoreType.DMA,
            pltpu.SemaphoreType.DMA,
        ],
    )(_sc_body)
    return k(ids_flat, word_emb, pos_emb, gamma, beta)


def kernel(input_ids, word_emb, pos_emb, gamma, beta):
    ids_flat = input_ids.reshape(-1).astype(jnp.int32)
    out = _run(ids_flat, word_emb, pos_emb, gamma, beta)
    return out.reshape(BATCH, SEQ, D_MODEL)


# speedup vs baseline: 1.0016x; 1.0016x over previous
import functools

import jax
import jax.numpy as jnp
from jax import lax
from jax.experimental import pallas as pl
from jax.experimental.pallas import tpu as pltpu
from jax.experimental.pallas import tpu_sc as plsc

VOCAB = 100000
D_MODEL = 1024
MAX_POS = 2048
BATCH = 4
SEQ = 2048
EPS = 1e-05

NC = 2
NS = 16
NW = NC * NS
G = 32
S_PER_W = SEQ // NW
NBLK = D_MODEL // 128
UNROLL = 8
ROWS = BATCH * SEQ


def _rsqrt(x):
    i = lax.bitcast_convert_type(x, jnp.int32)
    i = jnp.int32(0x5F3759DF) - lax.shift_right_arithmetic(i, 1)
    y = lax.bitcast_convert_type(i, jnp.float32)
    for _ in range(3):
        y = y * (jnp.float32(1.5) - jnp.float32(0.5) * x * y * y)
    return y


_GATHER_DNUMS = lax.GatherDimensionNumbers(
    offset_dims=(), collapsed_slice_dims=(0,), start_index_map=(0,))


def _take16(v, idx):
    return lax.gather(v, idx[:, None], _GATHER_DNUMS, (1,),
                      mode=lax.GatherScatterMode.PROMISE_IN_BOUNDS)


def _lane_sum(v):
    lanes = lax.iota(jnp.int32, 16)
    for sh in (8, 4, 2, 1):
        v = v + _take16(v, (lanes + sh) & 15)
    return v


def _sc_body(ids_hbm, wemb_hbm, pemb_hbm, gamma_hbm, beta_hbm, out_hbm,
             idx0, idx1, rows0, rows1, pos_v, gamma_v, beta_v, mscr, rscr,
             gsem0, gsem1, osem0, osem1):
    wid = lax.axis_index("s") * NC + lax.axis_index("c")
    s0 = wid * S_PER_W

    pltpu.sync_copy(gamma_hbm, gamma_v)
    pltpu.sync_copy(beta_hbm, beta_v)

    def gbase(g):
        return (g & 3) * SEQ + s0 + (g >> 2) * G

    def compute(rows_v):
        z = jnp.zeros((16,), jnp.float32)

        def row_sum(r, carry3):
            def col_sum(i, carry4):
                a, a2 = carry4
                d = i * 16
                x = rows_v[r, pl.ds(d, 16)] + pos_v[r, pl.ds(d, 16)]
                rows_v[r, pl.ds(d, 16)] = x
                return a + x, a2 + x * x

            acc, acc2 = plsc.parallel_loop(
                0, D_MODEL // 16, unroll=UNROLL, carry=(z, z))(col_sum)
            s1 = _lane_sum(acc)
            s2 = _lane_sum(acc2)
            mean = s1 * jnp.float32(1.0 / D_MODEL)
            var = s2 * jnp.float32(1.0 / D_MODEL) - mean * mean
            mscr[r, pl.ds(0, 16)] = mean
            rscr[r, pl.ds(0, 16)] = _rsqrt(var + jnp.float32(EPS))
            return carry3

        lax.fori_loop(0, G, row_sum, 0)

        for kb in range(NBLK):
            gv = [gamma_v[pl.ds(kb * 128 + u * 16, 16)] for u in range(8)]
            bv = [beta_v[pl.ds(kb * 128 + u * 16, 16)] for u in range(8)]

            def row_norm(r, gv=gv, bv=bv, kb=kb):
                mean_v = mscr[r, pl.ds(0, 16)]
                rstd_v = rscr[r, pl.ds(0, 16)]
                for u in range(8):
                    d = kb * 128 + u * 16
                    x = rows_v[r, pl.ds(d, 16)]
                    rows_v[r, pl.ds(d, 16)] = (
                        (x - mean_v) * rstd_v * gv[u] + bv[u])

            plsc.parallel_loop(0, G, unroll=4)(row_norm)

    pltpu.sync_copy(pemb_hbm.at[pl.ds(s0, G)], pos_v)
    pltpu.sync_copy(ids_hbm.at[pl.ds(s0, G)], idx0)
    pltpu.async_copy(wemb_hbm.at[idx0], rows0, gsem0)

    def pipe(t, carry):
        g0 = 2 * t
        g1 = 2 * t + 1
        base0 = gbase(g0)
        base1 = gbase(g1)

        @pl.when(t == 2)
        def _():
            pltpu.sync_copy(pemb_hbm.at[pl.ds(s0 + G, G)], pos_v)

        pltpu.make_async_copy(wemb_hbm.at[idx0], rows0, gsem0).wait()

        @pl.when(t > 0)
        def _():
            pltpu.make_async_copy(rows1, out_hbm.at[pl.ds(base1, G)],
                                  osem1).wait()

        pltpu.sync_copy(ids_hbm.at[pl.ds(base1, G)], idx1)
        pltpu.async_copy(wemb_hbm.at[idx1], rows1, gsem1)

        compute(rows0)
        pltpu.async_copy(rows0, out_hbm.at[pl.ds(base0, G)], osem0)

        pltpu.make_async_copy(wemb_hbm.at[idx1], rows1, gsem1).wait()

        @pl.when(t < 3)
        def _():
            base2 = gbase(g0 + 2)
            pltpu.make_async_copy(rows0, out_hbm.at[pl.ds(base2, G)],
                                  osem0).wait()
            pltpu.sync_copy(ids_hbm.at[pl.ds(base2, G)], idx0)
            pltpu.async_copy(wemb_hbm.at[idx0], rows0, gsem0)

        compute(rows1)
        pltpu.async_copy(rows1, out_hbm.at[pl.ds(base1, G)], osem1)
        return carry

    lax.fori_loop(0, BATCH, pipe, 0)

    last0 = gbase(jnp.int32(6))
    last1 = gbase(jnp.int32(7))
    pltpu.make_async_copy(rows0, out_hbm.at[pl.ds(last0, G)], osem0).wait()
    pltpu.make_async_copy(rows1, out_hbm.at[pl.ds(last1, G)], osem1).wait()


@jax.jit
def _run(ids_flat, word_emb, pos_emb, gamma, beta):
    mesh = plsc.VectorSubcoreMesh(core_axis_name="c", subcore_axis_name="s")
    k = functools.partial(
        pl.kernel,
        out_type=jax.ShapeDtypeStruct((ROWS, D_MODEL), jnp.float32),
        mesh=mesh,
        scratch_types=[
            pltpu.VMEM((G,), jnp.int32),
            pltpu.VMEM((G,), jnp.int32),
            pltpu.VMEM((G, D_MODEL), jnp.float32),
            pltpu.VMEM((G, D_MODEL), jnp.float32),
            pltpu.VMEM((G, D_MODEL), jnp.float32),
            pltpu.VMEM((D_MODEL,), jnp.float32),
            pltpu.VMEM((D_MODEL,), jnp.float32),
            pltpu.VMEM((G, 16), jnp.float32),
            pltpu.VMEM((G, 16), jnp.float32),
            pltpu.SemaphoreType.DMA,
            pltpu.SemaphoreType.DMA,
            pltpu.SemaphoreType.DMA,
            pltpu.SemaphoreType.DMA,
        ],
    )(_sc_body)
    return k(ids_flat, word_emb, pos_emb, gamma, beta)


def kernel(input_ids, word_emb, pos_emb, gamma, beta):
    ids_flat = input_ids.reshape(-1).astype(jnp.int32)
    out = _run(ids_flat, word_emb, pos_emb, gamma, beta)
    return out.reshape(BATCH, SEQ, D_MODEL)


# speedup vs baseline: 1.0818x; 1.0801x over previous
import functools

import jax
import jax.numpy as jnp
from jax import lax
from jax.experimental import pallas as pl
from jax.experimental.pallas import tpu as pltpu
from jax.experimental.pallas import tpu_sc as plsc

VOCAB = 100000
D_MODEL = 1024
MAX_POS = 2048
BATCH = 4
SEQ = 2048
EPS = 1e-05

NC = 2
NS = 16
NW = NC * NS
G = 32
S_PER_W = SEQ // NW
NBLK = D_MODEL // 128
UNROLL = 4
ROWS = BATCH * SEQ


def _rsqrt(x):
    i = lax.bitcast_convert_type(x, jnp.int32)
    i = jnp.int32(0x5F3759DF) - lax.shift_right_arithmetic(i, 1)
    y = lax.bitcast_convert_type(i, jnp.float32)
    for _ in range(3):
        y = y * (jnp.float32(1.5) - jnp.float32(0.5) * x * y * y)
    return y


_GATHER_DNUMS = lax.GatherDimensionNumbers(
    offset_dims=(), collapsed_slice_dims=(0,), start_index_map=(0,))


def _take16(v, idx):
    return lax.gather(v, idx[:, None], _GATHER_DNUMS, (1,),
                      mode=lax.GatherScatterMode.PROMISE_IN_BOUNDS)


def _lane_sum(v):
    lanes = lax.iota(jnp.int32, 16)
    for sh in (8, 4, 2, 1):
        v = v + _take16(v, (lanes + sh) & 15)
    return v


def _sc_body(ids_hbm, wemb_hbm, pemb_hbm, gamma_hbm, beta_hbm, out_hbm,
             idx0, idx1, rows0, rows1, pos_v, gamma_v, beta_v, mscr, rscr,
             gsem0, gsem1, osem0, osem1):
    wid = lax.axis_index("s") * NC + lax.axis_index("c")
    s0 = wid * S_PER_W

    pltpu.sync_copy(gamma_hbm, gamma_v)
    pltpu.sync_copy(beta_hbm, beta_v)

    def gbase(g):
        return (g & 3) * SEQ + s0 + (g >> 2) * G

    def compute(rows_v):
        z = jnp.zeros((16,), jnp.float32)

        def row_sum(r, carry3):
            def col_sum(i, carry4):
                a, a2 = carry4
                d = i * 16
                x = rows_v[r, pl.ds(d, 16)] + pos_v[r, pl.ds(d, 16)]
                rows_v[r, pl.ds(d, 16)] = x
                return a + x, a2 + x * x

            acc, acc2 = plsc.parallel_loop(
                0, D_MODEL // 16, unroll=UNROLL, carry=(z, z))(col_sum)
            s1 = _lane_sum(acc)
            s2 = _lane_sum(acc2)
            mean = s1 * jnp.float32(1.0 / D_MODEL)
            var = s2 * jnp.float32(1.0 / D_MODEL) - mean * mean
            mscr[r, pl.ds(0, 16)] = mean
            rscr[r, pl.ds(0, 16)] = _rsqrt(var + jnp.float32(EPS))
            return carry3

        lax.fori_loop(0, G, row_sum, 0)

        for kb in range(NBLK):
            gv = [gamma_v[pl.ds(kb * 128 + u * 16, 16)] for u in range(8)]
            bv = [beta_v[pl.ds(kb * 128 + u * 16, 16)] for u in range(8)]

            def row_norm(r, gv=gv, bv=bv, kb=kb):
                mean_v = mscr[r, pl.ds(0, 16)]
                rstd_v = rscr[r, pl.ds(0, 16)]
                for u in range(8):
                    d = kb * 128 + u * 16
                    x = rows_v[r, pl.ds(d, 16)]
                    rows_v[r, pl.ds(d, 16)] = (
                        (x - mean_v) * rstd_v * gv[u] + bv[u])

            plsc.parallel_loop(0, G, unroll=2)(row_norm)

    pltpu.sync_copy(pemb_hbm.at[pl.ds(s0, G)], pos_v)
    pltpu.sync_copy(ids_hbm.at[pl.ds(s0, G)], idx0)
    pltpu.async_copy(wemb_hbm.at[idx0], rows0, gsem0)

    def pipe(t, carry):
        g0 = 2 * t
        g1 = 2 * t + 1
        base0 = gbase(g0)
        base1 = gbase(g1)

        @pl.when(t == 2)
        def _():
            pltpu.sync_copy(pemb_hbm.at[pl.ds(s0 + G, G)], pos_v)

        pltpu.make_async_copy(wemb_hbm.at[idx0], rows0, gsem0).wait()

        @pl.when(t > 0)
        def _():
            pltpu.make_async_copy(rows1, out_hbm.at[pl.ds(base1, G)],
                                  osem1).wait()

        pltpu.sync_copy(ids_hbm.at[pl.ds(base1, G)], idx1)
        pltpu.async_copy(wemb_hbm.at[idx1], rows1, gsem1)

        compute(rows0)
        pltpu.async_copy(rows0, out_hbm.at[pl.ds(base0, G)], osem0)

        pltpu.make_async_copy(wemb_hbm.at[idx1], rows1, gsem1).wait()

        @pl.when(t < 3)
        def _():
            base2 = gbase(g0 + 2)
            pltpu.make_async_copy(rows0, out_hbm.at[pl.ds(base2, G)],
                                  osem0).wait()
            pltpu.sync_copy(ids_hbm.at[pl.ds(base2, G)], idx0)
            pltpu.async_copy(wemb_hbm.at[idx0], rows0, gsem0)

        compute(rows1)
        pltpu.async_copy(rows1, out_hbm.at[pl.ds(base1, G)], osem1)
        return carry

    lax.fori_loop(0, BATCH, pipe, 0)

    last0 = gbase(jnp.int32(6))
    last1 = gbase(jnp.int32(7))
    pltpu.make_async_copy(rows0, out_hbm.at[pl.ds(last0, G)], osem0).wait()
    pltpu.make_async_copy(rows1, out_hbm.at[pl.ds(last1, G)], osem1).wait()


@jax.jit
def _run(ids_flat, word_emb, pos_emb, gamma, beta):
    mesh = plsc.VectorSubcoreMesh(core_axis_name="c", subcore_axis_name="s")
    k = functools.partial(
        pl.kernel,
        out_type=jax.ShapeDtypeStruct((ROWS, D_MODEL), jnp.float32),
        mesh=mesh,
        scratch_types=[
            pltpu.VMEM((G,), jnp.int32),
            pltpu.VMEM((G,), jnp.int32),
            pltpu.VMEM((G, D_MODEL), jnp.float32),
            pltpu.VMEM((G, D_MODEL), jnp.float32),
            pltpu.VMEM((G, D_MODEL), jnp.float32),
            pltpu.VMEM((D_MODEL,), jnp.float32),
            pltpu.VMEM((D_MODEL,), jnp.float32),
            pltpu.VMEM((G, 16), jnp.float32),
            pltpu.VMEM((G, 16), jnp.float32),
            pltpu.SemaphoreType.DMA,
            pltpu.SemaphoreType.DMA,
            pltpu.SemaphoreType.DMA,
            pltpu.SemaphoreType.DMA,
        ],
    )(_sc_body)
    return k(ids_flat, word_emb, pos_emb, gamma, beta)


def kernel(input_ids, word_emb, pos_emb, gamma, beta):
    ids_flat = input_ids.reshape(-1).astype(jnp.int32)
    out = _run(ids_flat, word_emb, pos_emb, gamma, beta)
    return out.reshape(BATCH, SEQ, D_MODEL)


# speedup vs baseline: 1.5424x; 1.4259x over previous
import functools

import jax
import jax.numpy as jnp
from jax import lax
from jax.experimental import pallas as pl
from jax.experimental.pallas import tpu as pltpu
from jax.experimental.pallas import tpu_sc as plsc

VOCAB = 100000
D_MODEL = 1024
MAX_POS = 2048
BATCH = 4
SEQ = 2048
EPS = 1e-05

NC = 2
NS = 16
NW = NC * NS
G = 32
ROWS = BATCH * SEQ
NCHUNK = 2
CROWS = ROWS // NCHUNK
RPW = CROWS // NW
NPAIR = RPW // (2 * G)
SBLK = 256


def _sc_gather_body(ids_hbm, wemb_hbm, tok_hbm,
                    idx0, idx1, rows0, rows1, gsem0, gsem1, osem0, osem1):
    wid = lax.axis_index("s") * NC + lax.axis_index("c")
    row0 = wid * RPW

    pltpu.sync_copy(ids_hbm.at[pl.ds(row0, G)], idx0)
    pltpu.async_copy(wemb_hbm.at[idx0], rows0, gsem0)

    def pipe(t, carry):
        b0 = row0 + 2 * t * G
        b1 = b0 + G
        b2 = b0 + 2 * G

        pltpu.make_async_copy(wemb_hbm.at[idx0], rows0, gsem0).wait()

        @pl.when(t > 0)
        def _():
            pltpu.make_async_copy(rows1, tok_hbm.at[pl.ds(b1 - 2 * G, G)],
                                  osem1).wait()

        pltpu.sync_copy(ids_hbm.at[pl.ds(b1, G)], idx1)
        pltpu.async_copy(wemb_hbm.at[idx1], rows1, gsem1)
        pltpu.async_copy(rows0, tok_hbm.at[pl.ds(b0, G)], osem0)

        pltpu.make_async_copy(wemb_hbm.at[idx1], rows1, gsem1).wait()

        @pl.when(t < NPAIR - 1)
        def _():
            pltpu.make_async_copy(rows0, tok_hbm.at[pl.ds(b0, G)],
                                  osem0).wait()
            pltpu.sync_copy(ids_hbm.at[pl.ds(b2, G)], idx0)
            pltpu.async_copy(wemb_hbm.at[idx0], rows0, gsem0)

        pltpu.async_copy(rows1, tok_hbm.at[pl.ds(b1, G)], osem1)
        return carry

    lax.fori_loop(0, NPAIR, pipe, 0)

    lastb = row0 + RPW - 2 * G
    pltpu.make_async_copy(rows0, tok_hbm.at[pl.ds(lastb, G)], osem0).wait()
    pltpu.make_async_copy(rows1, tok_hbm.at[pl.ds(lastb + G, G)],
                          osem1).wait()


def _sc_gather(ids_chunk, word_emb):
    mesh = plsc.VectorSubcoreMesh(core_axis_name="c", subcore_axis_name="s")
    k = functools.partial(
        pl.kernel,
        out_type=jax.ShapeDtypeStruct((CROWS, D_MODEL), jnp.float32),
        mesh=mesh,
        scratch_types=[
            pltpu.VMEM((G,), jnp.int32),
            pltpu.VMEM((G,), jnp.int32),
            pltpu.VMEM((G, D_MODEL), jnp.float32),
            pltpu.VMEM((G, D_MODEL), jnp.float32),
            pltpu.SemaphoreType.DMA,
            pltpu.SemaphoreType.DMA,
            pltpu.SemaphoreType.DMA,
            pltpu.SemaphoreType.DMA,
        ],
    )(_sc_gather_body)
    return k(ids_chunk, word_emb)


def _tc_norm_body(tok_ref, pos_ref, gamma_ref, beta_ref, out_ref):
    x = tok_ref[...] + pos_ref[...]
    mean = jnp.mean(x, axis=1, keepdims=True)
    var = jnp.mean(jnp.square(x), axis=1, keepdims=True) - mean * mean
    y = (x - mean) * lax.rsqrt(var + EPS)
    out_ref[...] = y * gamma_ref[...] + beta_ref[...]


def _tc_norm(tok_chunk, pos_emb, gamma2, beta2, nb):
    ns = SEQ // SBLK
    return pl.pallas_call(
        _tc_norm_body,
        grid=(ns, nb),
        in_specs=[
            pl.BlockSpec((SBLK, D_MODEL), lambda si, bi: (bi * ns + si, 0)),
            pl.BlockSpec((SBLK, D_MODEL), lambda si, bi: (si, 0)),
            pl.BlockSpec((1, D_MODEL), lambda si, bi: (0, 0)),
            pl.BlockSpec((1, D_MODEL), lambda si, bi: (0, 0)),
        ],
        out_specs=pl.BlockSpec((SBLK, D_MODEL), lambda si, bi: (bi * ns + si, 0)),
        out_shape=jax.ShapeDtypeStruct((nb * SEQ, D_MODEL), jnp.float32),
    )(tok_chunk, pos_emb, gamma2, beta2)


@jax.jit
def _run(ids_flat, word_emb, pos_emb, gamma, beta):
    gamma2 = gamma.reshape(1, D_MODEL)
    beta2 = beta.reshape(1, D_MODEL)
    nb = CROWS // SEQ
    outs = []
    for c in range(NCHUNK):
        tok = _sc_gather(ids_flat[c * CROWS:(c + 1) * CROWS], word_emb)
        outs.append(_tc_norm(tok, pos_emb, gamma2, beta2, nb))
    return jnp.concatenate(outs, axis=0)


def kernel(input_ids, word_emb, pos_emb, gamma, beta):
    ids_flat = input_ids.reshape(-1).astype(jnp.int32)
    out = _run(ids_flat, word_emb, pos_emb, gamma, beta)
    return out.reshape(BATCH, SEQ, D_MODEL)


# speedup vs baseline: 1.6124x; 1.0454x over previous
import functools

import jax
import jax.numpy as jnp
from jax import lax
from jax.experimental import pallas as pl
from jax.experimental.pallas import tpu as pltpu
from jax.experimental.pallas import tpu_sc as plsc

VOCAB = 100000
D_MODEL = 1024
MAX_POS = 2048
BATCH = 4
SEQ = 2048
EPS = 1e-05

NC = 2
NS = 16
NW = NC * NS
G = 32
ROWS = BATCH * SEQ
NCHUNK = 2
CROWS = ROWS // NCHUNK
RPW = CROWS // NW
NPAIR = RPW // (2 * G)
SBLK = 512


def _sc_gather_body(ids_hbm, wemb_hbm, tok_hbm,
                    idx0, idx1, rows0, rows1, gsem0, gsem1, osem0, osem1):
    wid = lax.axis_index("s") * NC + lax.axis_index("c")
    row0 = wid * RPW

    pltpu.sync_copy(ids_hbm.at[pl.ds(row0, G)], idx0)
    pltpu.async_copy(wemb_hbm.at[idx0], rows0, gsem0)

    def pipe(t, carry):
        b0 = row0 + 2 * t * G
        b1 = b0 + G
        b2 = b0 + 2 * G

        pltpu.make_async_copy(wemb_hbm.at[idx0], rows0, gsem0).wait()

        @pl.when(t > 0)
        def _():
            pltpu.make_async_copy(rows1, tok_hbm.at[pl.ds(b1 - 2 * G, G)],
                                  osem1).wait()

        pltpu.sync_copy(ids_hbm.at[pl.ds(b1, G)], idx1)
        pltpu.async_copy(wemb_hbm.at[idx1], rows1, gsem1)
        pltpu.async_copy(rows0, tok_hbm.at[pl.ds(b0, G)], osem0)

        pltpu.make_async_copy(wemb_hbm.at[idx1], rows1, gsem1).wait()

        @pl.when(t < NPAIR - 1)
        def _():
            pltpu.make_async_copy(rows0, tok_hbm.at[pl.ds(b0, G)],
                                  osem0).wait()
            pltpu.sync_copy(ids_hbm.at[pl.ds(b2, G)], idx0)
            pltpu.async_copy(wemb_hbm.at[idx0], rows0, gsem0)

        pltpu.async_copy(rows1, tok_hbm.at[pl.ds(b1, G)], osem1)
        return carry

    lax.fori_loop(0, NPAIR, pipe, 0)

    lastb = row0 + RPW - 2 * G
    pltpu.make_async_copy(rows0, tok_hbm.at[pl.ds(lastb, G)], osem0).wait()
    pltpu.make_async_copy(rows1, tok_hbm.at[pl.ds(lastb + G, G)],
                          osem1).wait()


def _sc_gather(ids_chunk, word_emb):
    mesh = plsc.VectorSubcoreMesh(core_axis_name="c", subcore_axis_name="s")
    k = functools.partial(
        pl.kernel,
        out_type=jax.ShapeDtypeStruct((CROWS, D_MODEL), jnp.float32),
        mesh=mesh,
        scratch_types=[
            pltpu.VMEM((G,), jnp.int32),
            pltpu.VMEM((G,), jnp.int32),
            pltpu.VMEM((G, D_MODEL), jnp.float32),
            pltpu.VMEM((G, D_MODEL), jnp.float32),
            pltpu.SemaphoreType.DMA,
            pltpu.SemaphoreType.DMA,
            pltpu.SemaphoreType.DMA,
            pltpu.SemaphoreType.DMA,
        ],
    )(_sc_gather_body)
    return k(ids_chunk, word_emb)


def _tc_norm_body(tok_ref, pos_ref, gamma_ref, beta_ref, out_ref):
    x = tok_ref[...] + pos_ref[...]
    mean = jnp.mean(x, axis=1, keepdims=True)
    var = jnp.mean(jnp.square(x), axis=1, keepdims=True) - mean * mean
    y = (x - mean) * lax.rsqrt(var + EPS)
    out_ref[...] = y * gamma_ref[...] + beta_ref[...]


def _tc_norm(tok_chunk, pos_emb, gamma2, beta2, nb):
    ns = SEQ // SBLK
    return pl.pallas_call(
        _tc_norm_body,
        grid=(ns, nb),
        in_specs=[
            pl.BlockSpec((SBLK, D_MODEL), lambda si, bi: (bi * ns + si, 0)),
            pl.BlockSpec((SBLK, D_MODEL), lambda si, bi: (si, 0)),
            pl.BlockSpec((1, D_MODEL), lambda si, bi: (0, 0)),
            pl.BlockSpec((1, D_MODEL), lambda si, bi: (0, 0)),
        ],
        out_specs=pl.BlockSpec((SBLK, D_MODEL), lambda si, bi: (bi * ns + si, 0)),
        out_shape=jax.ShapeDtypeStruct((nb * SEQ, D_MODEL), jnp.float32),
    )(tok_chunk, pos_emb, gamma2, beta2)


@jax.jit
def _run(ids_flat, word_emb, pos_emb, gamma, beta):
    gamma2 = gamma.reshape(1, D_MODEL)
    beta2 = beta.reshape(1, D_MODEL)
    nb = CROWS // SEQ
    toks = [_sc_gather(ids_flat[c * CROWS:(c + 1) * CROWS], word_emb)
            for c in range(NCHUNK)]
    outs = [_tc_norm(tok, pos_emb, gamma2, beta2, nb) for tok in toks]
    return jnp.concatenate(outs, axis=0)


def kernel(input_ids, word_emb, pos_emb, gamma, beta):
    ids_flat = input_ids.reshape(-1).astype(jnp.int32)
    out = _run(ids_flat, word_emb, pos_emb, gamma, beta)
    return out.reshape(BATCH, SEQ, D_MODEL)
